# Initial kernel scaffold; baseline (speedup 1.0000x reference)
import jax
import jax.numpy as jnp
from jax.experimental import pallas as pl


def kernel(x, edge_index, theta1, W1, theta2, W2):
    raise NotImplementedError("write your pallas kernel here")


# speedup vs baseline: 4.9068x; 4.9068x over previous
import functools

import jax
import jax.numpy as jnp
from jax import lax
from jax.experimental import pallas as pl
from jax.experimental.pallas import tpu as pltpu
from jax.experimental.pallas import tpu_sc as plsc

N = 10000
E = 320000
D = 128
HID = 64
HEADS = 4
KORD = 8
C = 16

NT = 16
NC = 2
NW = NT * NC
N_PAD = 10240
RPT = N_PAD // NT
PAD_ROW = N
B = 128
E_PAD = 327680
EPW = E_PAD // NW
CPW = EPW // B
EPT = E_PAD // NT
CPT = EPT // B
RCH = 128

_f32 = jnp.float32


def _mesh():
    return plsc.VectorSubcoreMesh(core_axis_name="c", subcore_axis_name="s")


@functools.partial(
    pl.kernel,
    out_type=jax.ShapeDtypeStruct((2, N_PAD, 16), _f32),
    mesh=_mesh(),
    compiler_params=pltpu.CompilerParams(use_tc_tiling_on_sc=False),
    scratch_types=[
        pltpu.VMEM((B,), jnp.int32),
        pltpu.VMEM((B, 16), _f32),
        pltpu.VMEM((RPT, 16), _f32),
        pltpu.VMEM_SHARED((N_PAD, 16), _f32),
    ],
)
def _deg_kernel(src_hbm, dst_hbm, ones_hbm, z16_hbm, out_hbm, idx_v, ones_v,
                bounce, acc):
    c = lax.axis_index("c")
    s = lax.axis_index("s")
    pltpu.sync_copy(z16_hbm, bounce)
    pltpu.sync_copy(bounce, acc.at[pl.ds(s * RPT, RPT)])
    pltpu.sync_copy(ones_hbm, ones_v)
    plsc.subcore_barrier()

    def run(idx_hbm):
        def chunk(i, carry):
            base = s * EPT + i * B
            pltpu.sync_copy(idx_hbm.at[pl.ds(base, B)], idx_v)
            pltpu.sync_copy(ones_v, acc.at[idx_v], add=True)
            return carry
        lax.fori_loop(0, CPT, chunk, 0)

    @pl.when(c == 0)
    def _():
        run(src_hbm)

    @pl.when(c == 1)
    def _():
        run(dst_hbm)

    plsc.subcore_barrier()
    pltpu.sync_copy(acc.at[pl.ds(s * RPT, RPT)], bounce)
    pltpu.sync_copy(bounce, out_hbm.at[c].at[pl.ds(s * RPT, RPT)])


@functools.partial(
    pl.kernel,
    out_type=jax.ShapeDtypeStruct((NC, N_PAD, D), _f32),
    mesh=_mesh(),
    compiler_params=pltpu.CompilerParams(use_tc_tiling_on_sc=False),
    scratch_types=[
        pltpu.VMEM((B,), jnp.int32),
        pltpu.VMEM((B,), jnp.int32),
        pltpu.VMEM((B, D), _f32),
        pltpu.VMEM((RCH, D), _f32),
        pltpu.VMEM_SHARED((N_PAD, D), _f32),
        pltpu.SemaphoreType.DMA,
    ],
)
def _l1_gather_scatter(u_hbm, src_hbm, dst_hbm, z128_hbm, out_hbm,
                       idx_v, dst_v, rows_v, bounce, acc, sem):
    c = lax.axis_index("c")
    s = lax.axis_index("s")
    pltpu.sync_copy(z128_hbm.at[pl.ds(0, RCH)], bounce)
    for r in range(RPT // RCH):
        pltpu.sync_copy(bounce, acc.at[pl.ds(s * RPT + r * RCH, RCH)])
    plsc.subcore_barrier()

    wbase = (c * NT + s) * EPW

    def chunk(i, carry):
        base = wbase + i * B
        pltpu.sync_copy(src_hbm.at[pl.ds(base, B)], idx_v)
        pltpu.sync_copy(dst_hbm.at[pl.ds(base, B)], dst_v)
        pltpu.async_copy(u_hbm.at[idx_v], rows_v, sem).wait()
        pltpu.sync_copy(rows_v, acc.at[dst_v], add=True)
        return carry
    lax.fori_loop(0, CPW, chunk, 0)
    plsc.subcore_barrier()

    for r in range(RPT // RCH):
        rsl = pl.ds(s * RPT + r * RCH, RCH)
        pltpu.sync_copy(acc.at[rsl], bounce)
        pltpu.sync_copy(bounce, out_hbm.at[c].at[rsl])


@functools.partial(
    pl.kernel,
    out_type=jax.ShapeDtypeStruct((N_PAD, C), _f32),
    mesh=_mesh(),
    compiler_params=pltpu.CompilerParams(use_tc_tiling_on_sc=False),
    scratch_types=[
        pltpu.VMEM((B,), jnp.int32),
        pltpu.VMEM((B,), jnp.int32),
        pltpu.VMEM((B, C), _f32),
        pltpu.VMEM((8, C), _f32),
        pltpu.VMEM((RCH, C), _f32),
        pltpu.VMEM((RCH, C), _f32),
        pltpu.VMEM((RCH, C), _f32),
        pltpu.VMEM((RPT, C), _f32),
        pltpu.VMEM((RPT, C), _f32),
        pltpu.VMEM((RPT, C), _f32),
        pltpu.VMEM((RPT, C), _f32),
        pltpu.VMEM((RPT, C), _f32),
        pltpu.VMEM_SHARED((N_PAD, C), _f32),
        pltpu.VMEM_SHARED((N_PAD, C), _f32),
        pltpu.SemaphoreType.DMA,
    ],
)
def _l2_fused(g_hbm, u0_hbm, y0_hbm, th2_hbm, ab16_hbm, src_hbm, dst_hbm,
              z16_hbm, yout_hbm,
              idx_v, dst_v, rows_v, th2_v, pbuf, zbuf, unb, tna, tnb2,
              abuf, bbuf, ybuf, u_s, acc, sem):
    c = lax.axis_index("c")
    s = lax.axis_index("s")

    @pl.when(c == 0)
    def _():
        sl = pl.ds(s * RPT, RPT)
        pltpu.sync_copy(th2_hbm, th2_v)
        pltpu.sync_copy(z16_hbm.at[pl.ds(0, RCH)], zbuf)
        pltpu.sync_copy(g_hbm.at[sl], tna)
        pltpu.sync_copy(g_hbm.at[sl], tnb2)
        pltpu.sync_copy(u0_hbm.at[sl], ybuf)
        pltpu.sync_copy(ybuf, u_s.at[sl])
        pltpu.sync_copy(y0_hbm.at[sl], ybuf)
        pltpu.sync_copy(ab16_hbm.at[0].at[sl], abuf)
        pltpu.sync_copy(ab16_hbm.at[1].at[sl], bbuf)
        pltpu.sync_copy(z16_hbm.at[pl.ds(0, RCH)], zbuf)
        for r in range(RPT // RCH):
            pltpu.sync_copy(zbuf, acc.at[pl.ds(s * RPT + r * RCH, RCH)])
        plsc.subcore_barrier()

        bufs = [tna, tnb2]
        for k in range(1, KORD):
            alpha = 1.0 if k == 1 else 2.0
            beta = 0.0 if k == 1 else 1.0
            sub = bufs[(k - 1) % 2]

            def chunk(i, carry):
                base = s * EPT + i * B
                pltpu.sync_copy(src_hbm.at[pl.ds(base, B)], idx_v)
                pltpu.sync_copy(dst_hbm.at[pl.ds(base, B)], dst_v)
                pltpu.async_copy(u_s.at[idx_v], rows_v, sem).wait()
                pltpu.sync_copy(rows_v, acc.at[dst_v], add=True)
                return carry
            lax.fori_loop(0, CPT, chunk, 0)
            plsc.subcore_barrier()

            tcoef = th2_v[k, pl.ds(0, C)]
            for r in range(RPT // RCH):
                rsl = pl.ds(s * RPT + r * RCH, RCH)
                pltpu.sync_copy(acc.at[rsl], pbuf)
                pltpu.sync_copy(zbuf, acc.at[rsl])

                def rowop(i, carry):
                    t = r * RCH + i
                    p = pbuf[i, pl.ds(0, C)]
                    bb = bbuf[t, pl.ds(0, C)]
                    tn = (alpha * bb) * p
                    if beta != 0.0:
                        tn = tn - sub[t, pl.ds(0, C)]
                    sub[t, pl.ds(0, C)] = tn
                    unb[i, pl.ds(0, C)] = abuf[t, pl.ds(0, C)] * tn
                    ybuf[t, pl.ds(0, C)] = (ybuf[t, pl.ds(0, C)]
                                            + tcoef * tn)
                    return carry
                lax.fori_loop(0, RCH, rowop, 0)

                pltpu.sync_copy(unb, u_s.at[rsl])
            plsc.subcore_barrier()

        pltpu.sync_copy(ybuf, yout_hbm.at[pl.ds(s * RPT, RPT)])


_PBLK = 640


def _prescale_body(deg_ref, x_ref, u0_ref, ab16_ref):
    dg = deg_ref[...]
    a16 = lax.rsqrt(jnp.maximum(dg[0], 1.0))
    b16 = -lax.rsqrt(jnp.maximum(dg[1], 1.0))
    ab16_ref[0] = a16
    ab16_ref[1] = b16
    u0_ref[...] = x_ref[...] * a16[:, :1]


def _prescale(deg16, x_pad):
    return pl.pallas_call(
        _prescale_body,
        grid=(N_PAD // _PBLK,),
        in_specs=[
            pl.BlockSpec((2, _PBLK, 16), lambda i: (0, i, 0)),
            pl.BlockSpec((_PBLK, D), lambda i: (i, 0)),
        ],
        out_specs=[
            pl.BlockSpec((_PBLK, D), lambda i: (i, 0)),
            pl.BlockSpec((2, _PBLK, 16), lambda i: (0, i, 0)),
        ],
        out_shape=[
            jax.ShapeDtypeStruct((N_PAD, D), _f32),
            jax.ShapeDtypeStruct((2, N_PAD, 16), _f32),
        ],
    )(deg16, x_pad)


@functools.lru_cache(None)
def _combine(alpha, beta):
    def body(p_ref, tp_ref, ab_ref, tn_ref, un_ref):
        p = p_ref[0] + p_ref[1]
        a = ab_ref[0][:, :1]
        b = ab_ref[1][:, :1]
        tn = (alpha * b) * p
        if beta != 0.0:
            tn = tn - beta * tp_ref[...]
        tn_ref[...] = tn
        un_ref[...] = a * tn

    def call(p, t_prev, ab16):
        return pl.pallas_call(
            body,
            grid=(N_PAD // _PBLK,),
            in_specs=[
                pl.BlockSpec((2, _PBLK, D), lambda i: (0, i, 0)),
                pl.BlockSpec((_PBLK, D), lambda i: (i, 0)),
                pl.BlockSpec((2, _PBLK, 16), lambda i: (0, i, 0)),
            ],
            out_specs=[
                pl.BlockSpec((_PBLK, D), lambda i: (i, 0)),
                pl.BlockSpec((_PBLK, D), lambda i: (i, 0)),
            ],
            out_shape=[
                jax.ShapeDtypeStruct((N_PAD, D), _f32),
                jax.ShapeDtypeStruct((N_PAD, D), _f32),
            ],
        )(p, t_prev, ab16)
    return call


_MBLK = 1024


def _mm1_body(t_ref, w_ref, thr_ref, y_ref):
    w = w_ref[...]
    acc = jnp.zeros((_MBLK, HEADS * HID), _f32)
    for k in range(KORD):
        wk = w * thr_ref[k, :][None, :]
        acc = acc + jnp.dot(t_ref[k], wk, preferred_element_type=_f32)
    y_ref[...] = acc


def _mm1(tcat, w1r, thr):
    return pl.pallas_call(
        _mm1_body,
        grid=(N_PAD // _MBLK,),
        in_specs=[
            pl.BlockSpec((KORD, _MBLK, D), lambda i: (0, i, 0)),
            pl.BlockSpec((D, HEADS * HID), lambda i: (0, 0)),
            pl.BlockSpec((KORD, HEADS * HID), lambda i: (0, 0)),
        ],
        out_specs=pl.BlockSpec((_MBLK, HEADS * HID), lambda i: (i, 0)),
        out_shape=jax.ShapeDtypeStruct((N_PAD, HEADS * HID), _f32),
    )(tcat, w1r, thr)


_M2BLK = 2048


def _mm2_body(y_ref, w2_ref, th2_ref, ab16_ref, g_ref, u0_ref, y0_ref,
              th2x_ref):
    g = jnp.dot(y_ref[...], w2_ref[...], preferred_element_type=_f32)
    g_ref[...] = g
    u0_ref[...] = ab16_ref[0] * g
    y0_ref[...] = g * th2_ref[0, 0:1][None, :]
    th2x_ref[...] = jnp.broadcast_to(
        th2_ref[...].reshape(KORD, 1), (KORD, C))


def _mm2(y, w2v, th2, ab16):
    return pl.pallas_call(
        _mm2_body,
        grid=(N_PAD // _M2BLK,),
        in_specs=[
            pl.BlockSpec((_M2BLK, HEADS * HID), lambda i: (i, 0)),
            pl.BlockSpec((HEADS * HID, C), lambda i: (0, 0)),
            pl.BlockSpec((1, KORD), lambda i: (0, 0)),
            pl.BlockSpec((2, _M2BLK, 16), lambda i: (0, i, 0)),
        ],
        out_specs=[
            pl.BlockSpec((_M2BLK, C), lambda i: (i, 0)),
            pl.BlockSpec((_M2BLK, C), lambda i: (i, 0)),
            pl.BlockSpec((_M2BLK, C), lambda i: (i, 0)),
            pl.BlockSpec((KORD, C), lambda i: (0, 0)),
        ],
        out_shape=[
            jax.ShapeDtypeStruct((N_PAD, C), _f32),
            jax.ShapeDtypeStruct((N_PAD, C), _f32),
            jax.ShapeDtypeStruct((N_PAD, C), _f32),
            jax.ShapeDtypeStruct((KORD, C), _f32),
        ],
    )(y, w2v, th2, ab16)


def _act_body(y_ref, o_ref):
    v = y_ref[...]
    e = jnp.where(v > 0, v, jnp.exp(jnp.minimum(v, 0.0)) - 1.0)
    m = jnp.max(e, axis=1, keepdims=True)
    z = e - m
    lse = jnp.log(jnp.sum(jnp.exp(z), axis=1, keepdims=True))
    o_ref[...] = z - lse


def _act(y):
    return pl.pallas_call(
        _act_body,
        grid=(N_PAD // _M2BLK,),
        in_specs=[pl.BlockSpec((_M2BLK, C), lambda i: (i, 0))],
        out_specs=pl.BlockSpec((_M2BLK, C), lambda i: (i, 0)),
        out_shape=jax.ShapeDtypeStruct((N_PAD, C), _f32),
    )(y)


def kernel(x, edge_index, theta1, W1, theta2, W2):
    src = edge_index[0]
    dst = edge_index[1]
    pad_i = jnp.full((E_PAD - E,), PAD_ROW, jnp.int32)
    src_p = jnp.concatenate([src, pad_i])
    dst_p = jnp.concatenate([dst, pad_i])
    x_pad = jnp.pad(x, ((0, N_PAD - N), (0, 0)))

    ones16 = jnp.ones((B, 16), _f32)
    z16r = jnp.zeros((RPT, 16), _f32)
    z128 = jnp.zeros((RPT, D), _f32)

    deg16 = _deg_kernel(src_p, dst_p, ones16, z16r)
    u0, ab16 = _prescale(deg16, x_pad)

    ts = [x_pad]
    u = u0
    for k in range(1, KORD):
        al, be = (1.0, 0.0) if k == 1 else (2.0, 1.0)
        t_pp = ts[-2] if k >= 2 else ts[-1]
        p = _l1_gather_scatter(u, src_p, dst_p, z128)
        t_new, u_next = _combine(al, be)(p, t_pp, ab16)
        ts.append(t_new)
        u = u_next

    tcat = jnp.stack(ts)
    w1r = jnp.transpose(W1, (1, 0, 2)).reshape(D, HEADS * HID)
    thr = jnp.repeat(theta1.T, HID, axis=1)
    y1 = _mm1(tcat, w1r, thr)

    g, u0p, y0, th2x = _mm2(y1, W2[0], theta2, ab16)
    yacc = _l2_fused(g, u0p, y0, th2x, ab16, src_p, dst_p, z16r)

    out = _act(yacc)
    return out[:N]


# speedup vs baseline: 9.8380x; 2.0050x over previous
import functools

import jax
import jax.numpy as jnp
from jax import lax
from jax.experimental import pallas as pl
from jax.experimental.pallas import tpu as pltpu
from jax.experimental.pallas import tpu_sc as plsc

N = 10000
E = 320000
D = 128
HID = 64
HEADS = 4
KORD = 8
C = 16

NT = 16
NC = 2
NW = NT * NC
N_PAD = 10240
RPT = N_PAD // NT
PAD_ROW = N
B = 128
E_PAD = 327680
NCHUNK = E_PAD // B
EPW = E_PAD // NW
CPW = EPW // B
EPT = E_PAD // NT
CPT = EPT // B
RCH = 128
NB1 = 4
NB2 = 4


def _pipelined_edge_loop(nchunks, nb, fire_gather, fire_scatter):
    def group(g, carry):
        def fire(b, cc):
            fire_gather(g * nb + b, b).start()
            return cc
        lax.fori_loop(0, nb, fire, 0)

        def scat(b, cc):
            ci = g * nb + b
            fire_gather(ci, b).wait()
            fire_scatter(ci, b).start(add=True)
            return cc
        lax.fori_loop(0, nb, scat, 0)

        def drain(b, cc):
            fire_scatter(g * nb + b, b).wait()
            return cc
        lax.fori_loop(0, nb, drain, 0)
        return carry
    lax.fori_loop(0, nchunks // nb, group, 0)

_f32 = jnp.float32


def _mesh():
    return plsc.VectorSubcoreMesh(core_axis_name="c", subcore_axis_name="s")


_SC_PARAMS = pltpu.CompilerParams(use_tc_tiling_on_sc=False)


@functools.partial(
    pl.kernel,
    out_type=jax.ShapeDtypeStruct((2, N_PAD, 16), _f32),
    mesh=_mesh(),
    compiler_params=_SC_PARAMS,
    scratch_types=[
        pltpu.VMEM((CPT, B), jnp.int32),
        pltpu.VMEM((B, 16), _f32),
        pltpu.VMEM((RPT, 16), _f32),
        pltpu.VMEM_SHARED((N_PAD, 16), _f32),
        pltpu.SemaphoreType.DMA((NB2,)),
    ],
)
def _deg_kernel(src_hbm, dst_hbm, ones_hbm, z16_hbm, out_hbm, slab, ones_v,
                bounce, acc, sems):
    c = lax.axis_index("c")
    s = lax.axis_index("s")
    pltpu.sync_copy(z16_hbm, bounce)
    pltpu.sync_copy(bounce, acc.at[pl.ds(s * RPT, RPT)])
    pltpu.sync_copy(ones_hbm, ones_v)

    @pl.when(c == 0)
    def _():
        pltpu.sync_copy(src_hbm.at[pl.ds(s * CPT, CPT)], slab)

    @pl.when(c == 1)
    def _():
        pltpu.sync_copy(dst_hbm.at[pl.ds(s * CPT, CPT)], slab)

    plsc.subcore_barrier()

    def scat(ci, b):
        return pltpu.make_async_copy(ones_v, acc.at[slab.at[ci]], sems.at[b])

    def group(g, carry):
        def fire(b, cc):
            scat(g * NB2 + b, b).start(add=True)
            return cc
        lax.fori_loop(0, NB2, fire, 0)

        def drain(b, cc):
            scat(g * NB2 + b, b).wait()
            return cc
        lax.fori_loop(0, NB2, drain, 0)
        return carry
    lax.fori_loop(0, CPT // NB2, group, 0)

    plsc.subcore_barrier()
    pltpu.sync_copy(acc.at[pl.ds(s * RPT, RPT)], bounce)
    pltpu.sync_copy(bounce, out_hbm.at[c].at[pl.ds(s * RPT, RPT)])


W1SC = 64

@functools.partial(
    pl.kernel,
    out_type=jax.ShapeDtypeStruct((NC, N_PAD, W1SC), _f32),
    mesh=_mesh(),
    compiler_params=_SC_PARAMS,
    scratch_types=[
        pltpu.VMEM((CPT, B), jnp.int32),
        pltpu.VMEM((CPT, B), jnp.int32),
        pltpu.VMEM((RCH, W1SC), _f32),
        pltpu.VMEM((NB1, B, W1SC), _f32),
        pltpu.SemaphoreType.DMA((NB1,)),
        pltpu.SemaphoreType.DMA((NB1,)),
        pltpu.VMEM_SHARED((N_PAD, W1SC), _f32),
    ],
)
def _l1_gather_scatter(u_hbm, src_hbm, dst_hbm, z64_hbm, out_hbm,
                       sslab, dslab, bounce, rows, gsems, ssems, acc):
    c = lax.axis_index("c")
    s = lax.axis_index("s")
    pltpu.sync_copy(z64_hbm.at[pl.ds(0, RCH)], bounce)
    for r in range(RPT // RCH):
        pltpu.sync_copy(bounce, acc.at[pl.ds(s * RPT + r * RCH, RCH)])
    pltpu.sync_copy(src_hbm.at[pl.ds(s * CPT, CPT)], sslab)
    pltpu.sync_copy(dst_hbm.at[pl.ds(s * CPT, CPT)], dslab)
    plsc.subcore_barrier()

    def gat(ci, b):
        return pltpu.make_async_copy(
            u_hbm.at[c].at[sslab.at[ci]], rows.at[b], gsems.at[b])

    def scat(ci, b):
        return pltpu.make_async_copy(
            rows.at[b], acc.at[dslab.at[ci]], ssems.at[b])

    _pipelined_edge_loop(CPT, NB1, gat, scat)
    plsc.subcore_barrier()

    for r in range(RPT // RCH):
        rsl = pl.ds(s * RPT + r * RCH, RCH)
        pltpu.sync_copy(acc.at[rsl], bounce)
        pltpu.sync_copy(bounce, out_hbm.at[c].at[rsl])


@functools.partial(
    pl.kernel,
    out_type=jax.ShapeDtypeStruct((N_PAD, C), _f32),
    mesh=_mesh(),
    compiler_params=_SC_PARAMS,
    scratch_types=[
        pltpu.VMEM((CPT, B), jnp.int32),
        pltpu.VMEM((CPT, B), jnp.int32),
        pltpu.VMEM((8, C), _f32),
        pltpu.VMEM((RCH, C), _f32),
        pltpu.VMEM((RCH, C), _f32),
        pltpu.VMEM((RCH, C), _f32),
        pltpu.VMEM((RPT, C), _f32),
        pltpu.VMEM((RPT, C), _f32),
        pltpu.VMEM((RPT, C), _f32),
        pltpu.VMEM((RPT, C), _f32),
        pltpu.VMEM((RPT, C), _f32),
        pltpu.VMEM((NB2, B, C), _f32),
        pltpu.SemaphoreType.DMA((NB2,)),
        pltpu.SemaphoreType.DMA((NB2,)),
        pltpu.VMEM_SHARED((N_PAD, C), _f32),
        pltpu.VMEM_SHARED((N_PAD, C), _f32),
    ],
)
def _l2_fused(g_hbm, u0_hbm, y0_hbm, th2_hbm, ab16_hbm, src_hbm, dst_hbm,
              z16_hbm, yout_hbm,
              sslab, dslab, th2_v, pbuf, zbuf, unb, tna, tnb2,
              abuf, bbuf, ybuf, rows, gsems, ssems, u_s, acc):
    c = lax.axis_index("c")
    s = lax.axis_index("s")

    @pl.when(c == 0)
    def _():
        sl = pl.ds(s * RPT, RPT)
        pltpu.sync_copy(th2_hbm, th2_v)
        pltpu.sync_copy(z16_hbm.at[pl.ds(0, RCH)], zbuf)
        pltpu.sync_copy(g_hbm.at[sl], tna)
        pltpu.sync_copy(g_hbm.at[sl], tnb2)
        pltpu.sync_copy(u0_hbm.at[sl], ybuf)
        pltpu.sync_copy(ybuf, u_s.at[sl])
        pltpu.sync_copy(y0_hbm.at[sl], ybuf)
        pltpu.sync_copy(ab16_hbm.at[0].at[sl], abuf)
        pltpu.sync_copy(ab16_hbm.at[1].at[sl], bbuf)
        pltpu.sync_copy(src_hbm.at[pl.ds(s * CPT, CPT)], sslab)
        pltpu.sync_copy(dst_hbm.at[pl.ds(s * CPT, CPT)], dslab)
        for r in range(RPT // RCH):
            pltpu.sync_copy(zbuf, acc.at[pl.ds(s * RPT + r * RCH, RCH)])
        plsc.subcore_barrier()

        bufs = [tna, tnb2]
        for k in range(1, KORD):
            alpha = 1.0 if k == 1 else 2.0
            beta = 0.0 if k == 1 else 1.0
            sub = bufs[(k - 1) % 2]

            def gat(ci, b):
                return pltpu.make_async_copy(
                    u_s.at[sslab.at[ci]], rows.at[b], gsems.at[b])

            def scat(ci, b):
                return pltpu.make_async_copy(
                    rows.at[b], acc.at[dslab.at[ci]], ssems.at[b])

            _pipelined_edge_loop(CPT, NB2, gat, scat)
            plsc.subcore_barrier()

            tcoef = th2_v[k, pl.ds(0, C)]
            for r in range(RPT // RCH):
                rsl = pl.ds(s * RPT + r * RCH, RCH)
                pltpu.sync_copy(acc.at[rsl], pbuf)
                pltpu.sync_copy(zbuf, acc.at[rsl])

                def rowop(i, carry):
                    t = r * RCH + i
                    p = pbuf[i, pl.ds(0, C)]
                    bb = bbuf[t, pl.ds(0, C)]
                    tn = (alpha * bb) * p
                    if beta != 0.0:
                        tn = tn - sub[t, pl.ds(0, C)]
                    sub[t, pl.ds(0, C)] = tn
                    unb[i, pl.ds(0, C)] = abuf[t, pl.ds(0, C)] * tn
                    ybuf[t, pl.ds(0, C)] = (ybuf[t, pl.ds(0, C)]
                                            + tcoef * tn)
                    return carry
                lax.fori_loop(0, RCH, rowop, 0)

                pltpu.sync_copy(unb, u_s.at[rsl])
            plsc.subcore_barrier()

        pltpu.sync_copy(ybuf, yout_hbm.at[pl.ds(s * RPT, RPT)])


_PBLK = 640


def _prescale_body(deg_ref, x_ref, u0_ref, ab16_ref):
    dg = deg_ref[...]
    a16 = lax.rsqrt(jnp.maximum(dg[0], 1.0))
    b16 = -lax.rsqrt(jnp.maximum(dg[1], 1.0))
    ab16_ref[0] = a16
    ab16_ref[1] = b16
    a1 = a16[:, :1]
    x = x_ref[...]
    u0_ref[0] = x[:, :W1SC] * a1
    u0_ref[1] = x[:, W1SC:] * a1


def _prescale(deg16, x_pad):
    return pl.pallas_call(
        _prescale_body,
        grid=(N_PAD // _PBLK,),
        in_specs=[
            pl.BlockSpec((2, _PBLK, 16), lambda i: (0, i, 0)),
            pl.BlockSpec((_PBLK, D), lambda i: (i, 0)),
        ],
        out_specs=[
            pl.BlockSpec((2, _PBLK, W1SC), lambda i: (0, i, 0)),
            pl.BlockSpec((2, _PBLK, 16), lambda i: (0, i, 0)),
        ],
        out_shape=[
            jax.ShapeDtypeStruct((NC, N_PAD, W1SC), _f32),
            jax.ShapeDtypeStruct((2, N_PAD, 16), _f32),
        ],
    )(deg16, x_pad)


@functools.lru_cache(None)
def _combine(alpha, beta):
    def body(p_ref, tp_ref, ab_ref, tn_ref, un_ref):
        p = p_ref[...]
        a = ab_ref[0][:, :1][None]
        b = ab_ref[1][:, :1][None]
        tn = (alpha * b) * p
        if beta != 0.0:
            tn = tn - beta * tp_ref[...]
        tn_ref[...] = tn
        un_ref[...] = a * tn

    def call(p, t_prev, ab16):
        return pl.pallas_call(
            body,
            grid=(N_PAD // _PBLK,),
            in_specs=[
                pl.BlockSpec((2, _PBLK, W1SC), lambda i: (0, i, 0)),
                pl.BlockSpec((2, _PBLK, W1SC), lambda i: (0, i, 0)),
                pl.BlockSpec((2, _PBLK, 16), lambda i: (0, i, 0)),
            ],
            out_specs=[
                pl.BlockSpec((2, _PBLK, W1SC), lambda i: (0, i, 0)),
                pl.BlockSpec((2, _PBLK, W1SC), lambda i: (0, i, 0)),
            ],
            out_shape=[
                jax.ShapeDtypeStruct((NC, N_PAD, W1SC), _f32),
                jax.ShapeDtypeStruct((NC, N_PAD, W1SC), _f32),
            ],
        )(p, t_prev, ab16)
    return call


_MBLK = 1024


def _mm1_body(t_ref, w_ref, thr_ref, y_ref):
    w = w_ref[...]
    acc = jnp.zeros((_MBLK, HEADS * HID), _f32)
    for k in range(KORD):
        wk = w * thr_ref[k, :][None, :]
        acc = acc + jnp.dot(t_ref[k], wk, preferred_element_type=_f32)
    y_ref[...] = acc


def _mm1(tcat, w1r, thr):
    return pl.pallas_call(
        _mm1_body,
        grid=(N_PAD // _MBLK,),
        in_specs=[
            pl.BlockSpec((KORD, _MBLK, D), lambda i: (0, i, 0)),
            pl.BlockSpec((D, HEADS * HID), lambda i: (0, 0)),
            pl.BlockSpec((KORD, HEADS * HID), lambda i: (0, 0)),
        ],
        out_specs=pl.BlockSpec((_MBLK, HEADS * HID), lambda i: (i, 0)),
        out_shape=jax.ShapeDtypeStruct((N_PAD, HEADS * HID), _f32),
    )(tcat, w1r, thr)


_M2BLK = 2048


def _mm2_body(y_ref, w2_ref, th2_ref, ab16_ref, g_ref, u0_ref, y0_ref,
              th2x_ref):
    g = jnp.dot(y_ref[...], w2_ref[...], preferred_element_type=_f32)
    g_ref[...] = g
    u0_ref[...] = ab16_ref[0] * g
    y0_ref[...] = g * th2_ref[0, 0:1][None, :]
    th2x_ref[...] = jnp.broadcast_to(
        th2_ref[...].reshape(KORD, 1), (KORD, C))


def _mm2(y, w2v, th2, ab16):
    return pl.pallas_call(
        _mm2_body,
        grid=(N_PAD // _M2BLK,),
        in_specs=[
            pl.BlockSpec((_M2BLK, HEADS * HID), lambda i: (i, 0)),
            pl.BlockSpec((HEADS * HID, C), lambda i: (0, 0)),
            pl.BlockSpec((1, KORD), lambda i: (0, 0)),
            pl.BlockSpec((2, _M2BLK, 16), lambda i: (0, i, 0)),
        ],
        out_specs=[
            pl.BlockSpec((_M2BLK, C), lambda i: (i, 0)),
            pl.BlockSpec((_M2BLK, C), lambda i: (i, 0)),
            pl.BlockSpec((_M2BLK, C), lambda i: (i, 0)),
            pl.BlockSpec((KORD, C), lambda i: (0, 0)),
        ],
        out_shape=[
            jax.ShapeDtypeStruct((N_PAD, C), _f32),
            jax.ShapeDtypeStruct((N_PAD, C), _f32),
            jax.ShapeDtypeStruct((N_PAD, C), _f32),
            jax.ShapeDtypeStruct((KORD, C), _f32),
        ],
    )(y, w2v, th2, ab16)


def _act_body(y_ref, o_ref):
    v = y_ref[...]
    e = jnp.where(v > 0, v, jnp.exp(jnp.minimum(v, 0.0)) - 1.0)
    m = jnp.max(e, axis=1, keepdims=True)
    z = e - m
    lse = jnp.log(jnp.sum(jnp.exp(z), axis=1, keepdims=True))
    o_ref[...] = z - lse


def _act(y):
    return pl.pallas_call(
        _act_body,
        grid=(N_PAD // _M2BLK,),
        in_specs=[pl.BlockSpec((_M2BLK, C), lambda i: (i, 0))],
        out_specs=pl.BlockSpec((_M2BLK, C), lambda i: (i, 0)),
        out_shape=jax.ShapeDtypeStruct((N_PAD, C), _f32),
    )(y)


def kernel(x, edge_index, theta1, W1, theta2, W2):
    src = edge_index[0]
    dst = edge_index[1]
    pad_i = jnp.full((E_PAD - E,), PAD_ROW, jnp.int32)
    src_p = jnp.concatenate([src, pad_i]).reshape(NCHUNK, B)
    dst_p = jnp.concatenate([dst, pad_i]).reshape(NCHUNK, B)
    x_pad = jnp.pad(x, ((0, N_PAD - N), (0, 0)))

    ones16 = jnp.ones((B, 16), _f32)
    z16r = jnp.zeros((RPT, 16), _f32)
    z64 = jnp.zeros((RPT, W1SC), _f32)

    deg16 = _deg_kernel(src_p, dst_p, ones16, z16r)
    u0, ab16 = _prescale(deg16, x_pad)

    x_split = x_pad.reshape(N_PAD, NC, W1SC).transpose(1, 0, 2)
    ts = [x_split]
    u = u0
    for k in range(1, KORD):
        al, be = (1.0, 0.0) if k == 1 else (2.0, 1.0)
        t_pp = ts[-2] if k >= 2 else ts[-1]
        p = _l1_gather_scatter(u, src_p, dst_p, z64)
        t_new, u_next = _combine(al, be)(p, t_pp, ab16)
        ts.append(t_new)
        u = u_next

    tcat = jnp.stack(
        [x_pad] + [jnp.concatenate([t[0], t[1]], axis=1) for t in ts[1:]])
    w1r = jnp.transpose(W1, (1, 0, 2)).reshape(D, HEADS * HID)
    thr = jnp.repeat(theta1.T, HID, axis=1)
    y1 = _mm1(tcat, w1r, thr)

    g, u0p, y0, th2x = _mm2(y1, W2[0], theta2, ab16)
    yacc = _l2_fused(g, u0p, y0, th2x, ab16, src_p, dst_p, z16r)

    out = _act(yacc)
    return out[:N]


# speedup vs baseline: 10.8899x; 1.1069x over previous
import functools

import jax
import jax.numpy as jnp
from jax import lax
from jax.experimental import pallas as pl
from jax.experimental.pallas import tpu as pltpu
from jax.experimental.pallas import tpu_sc as plsc

N = 10000
E = 320000
D = 128
HID = 64
HEADS = 4
KORD = 8
C = 16

NT = 16
NC = 2
NW = NT * NC
N_PAD = 10240
RPT = N_PAD // NT
PAD_ROW = N
B = 128
E_PAD = 327680
NCHUNK = E_PAD // B
EPW = E_PAD // NW
CPW = EPW // B
EPT = E_PAD // NT
CPT = EPT // B
RCH = 128
NB1 = 4
NB2 = 4
NL2 = 2


def _ring_edge_loop(nchunks, nb, gat, scat):
    def step(ci, carry):
        b = lax.rem(ci, nb)

        @pl.when(ci >= nb)
        def _():
            scat(ci - nb, b).wait()
        gat(ci, b).start()

        @pl.when(ci >= 1)
        def _():
            b1 = lax.rem(ci - 1, nb)
            gat(ci - 1, b1).wait()
            scat(ci - 1, b1).start(add=True)
        return carry
    lax.fori_loop(0, nchunks, step, 0)
    last = nchunks - 1
    gat(last, last % nb).wait()
    scat(last, last % nb).start(add=True)
    for j in range(max(last - nb + 1, 0), last + 1):
        scat(j, j % nb).wait()


_f32 = jnp.float32


def _mesh():
    return plsc.VectorSubcoreMesh(core_axis_name="c", subcore_axis_name="s")


_SC_PARAMS = pltpu.CompilerParams(use_tc_tiling_on_sc=False)


@functools.partial(
    pl.kernel,
    out_type=jax.ShapeDtypeStruct((2, N_PAD, 16), _f32),
    mesh=_mesh(),
    compiler_params=_SC_PARAMS,
    scratch_types=[
        pltpu.VMEM((CPT, B), jnp.int32),
        pltpu.VMEM((B, 16), _f32),
        pltpu.VMEM((RPT, 16), _f32),
        pltpu.VMEM_SHARED((N_PAD, 16), _f32),
        pltpu.SemaphoreType.DMA((NB2,)),
    ],
)
def _deg_kernel(src_hbm, dst_hbm, ones_hbm, z16_hbm, out_hbm, slab, ones_v,
                bounce, acc, sems):
    c = lax.axis_index("c")
    s = lax.axis_index("s")
    pltpu.sync_copy(z16_hbm, bounce)
    pltpu.sync_copy(bounce, acc.at[pl.ds(s * RPT, RPT)])
    pltpu.sync_copy(ones_hbm, ones_v)

    @pl.when(c == 0)
    def _():
        pltpu.sync_copy(src_hbm.at[pl.ds(s * CPT, CPT)], slab)

    @pl.when(c == 1)
    def _():
        pltpu.sync_copy(dst_hbm.at[pl.ds(s * CPT, CPT)], slab)

    plsc.subcore_barrier()

    def scat(ci, b):
        return pltpu.make_async_copy(ones_v, acc.at[slab.at[ci]], sems.at[b])

    def step(ci, carry):
        b = lax.rem(ci, NB2)

        @pl.when(ci >= NB2)
        def _():
            scat(ci - NB2, b).wait()
        scat(ci, b).start(add=True)
        return carry
    lax.fori_loop(0, CPT, step, 0)
    for t in range(NB2):
        scat(CPT - NB2 + t, (CPT - NB2 + t) % NB2).wait()

    plsc.subcore_barrier()
    pltpu.sync_copy(acc.at[pl.ds(s * RPT, RPT)], bounce)
    pltpu.sync_copy(bounce, out_hbm.at[c].at[pl.ds(s * RPT, RPT)])


W1SC = 64

@functools.partial(
    pl.kernel,
    out_type=jax.ShapeDtypeStruct((NC, N_PAD, W1SC), _f32),
    mesh=_mesh(),
    compiler_params=_SC_PARAMS,
    scratch_types=[
        pltpu.VMEM((CPT, B), jnp.int32),
        pltpu.VMEM((CPT, B), jnp.int32),
        pltpu.VMEM((RCH, W1SC), _f32),
        pltpu.VMEM((NB1, B, W1SC), _f32),
        pltpu.SemaphoreType.DMA((NB1,)),
        pltpu.SemaphoreType.DMA((NB1,)),
        pltpu.VMEM_SHARED((N_PAD, W1SC), _f32),
    ],
)
def _l1_gather_scatter(u_hbm, src_hbm, dst_hbm, z64_hbm, out_hbm,
                       sslab, dslab, bounce, rows, gsems, ssems, acc):
    c = lax.axis_index("c")
    s = lax.axis_index("s")
    pltpu.sync_copy(z64_hbm.at[pl.ds(0, RCH)], bounce)
    for r in range(RPT // RCH):
        pltpu.sync_copy(bounce, acc.at[pl.ds(s * RPT + r * RCH, RCH)])
    pltpu.sync_copy(src_hbm.at[pl.ds(s * CPT, CPT)], sslab)
    pltpu.sync_copy(dst_hbm.at[pl.ds(s * CPT, CPT)], dslab)
    plsc.subcore_barrier()

    def gat(ci, b):
        return pltpu.make_async_copy(
            u_hbm.at[c].at[sslab.at[ci]], rows.at[b], gsems.at[b])

    def scat(ci, b):
        return pltpu.make_async_copy(
            rows.at[b], acc.at[dslab.at[ci]], ssems.at[b])

    _ring_edge_loop(CPT, NB1, gat, scat)
    plsc.subcore_barrier()

    for r in range(RPT // RCH):
        rsl = pl.ds(s * RPT + r * RCH, RCH)
        pltpu.sync_copy(acc.at[rsl], bounce)
        pltpu.sync_copy(bounce, out_hbm.at[c].at[rsl])


@functools.partial(
    pl.kernel,
    out_type=jax.ShapeDtypeStruct((N_PAD, C), _f32),
    mesh=_mesh(),
    compiler_params=_SC_PARAMS,
    scratch_types=[
        pltpu.VMEM((CPT, B), jnp.int32),
        pltpu.VMEM((CPT, B), jnp.int32),
        pltpu.VMEM((8, C), _f32),
        pltpu.VMEM((RCH, C), _f32),
        pltpu.VMEM((RCH, C), _f32),
        pltpu.VMEM((RCH, C), _f32),
        pltpu.VMEM((RPT, C), _f32),
        pltpu.VMEM((RPT, C), _f32),
        pltpu.VMEM((RPT, C), _f32),
        pltpu.VMEM((RPT, C), _f32),
        pltpu.VMEM((RPT, C), _f32),
        pltpu.VMEM((NL2, B, C), _f32),
        pltpu.SemaphoreType.DMA((NL2,)),
        pltpu.SemaphoreType.DMA((NL2,)),
        pltpu.VMEM_SHARED((N_PAD, C), _f32),
        pltpu.VMEM_SHARED((N_PAD, C), _f32),
    ],
)
def _l2_fused(g_hbm, u0_hbm, y0_hbm, th2_hbm, ab16_hbm, src_hbm, dst_hbm,
              z16_hbm, yout_hbm,
              sslab, dslab, th2_v, pbuf, zbuf, unb, tna, tnb2,
              abuf, bbuf, ybuf, rows, gsems, ssems, u_s, acc):
    c = lax.axis_index("c")
    s = lax.axis_index("s")

    @pl.when(c == 0)
    def _():
        sl = pl.ds(s * RPT, RPT)
        pltpu.sync_copy(th2_hbm, th2_v)
        pltpu.sync_copy(z16_hbm.at[pl.ds(0, RCH)], zbuf)
        pltpu.sync_copy(g_hbm.at[sl], tna)
        pltpu.sync_copy(g_hbm.at[sl], tnb2)
        pltpu.sync_copy(u0_hbm.at[sl], ybuf)
        pltpu.sync_copy(ybuf, u_s.at[sl])
        pltpu.sync_copy(y0_hbm.at[sl], ybuf)
        pltpu.sync_copy(ab16_hbm.at[0].at[sl], abuf)
        pltpu.sync_copy(ab16_hbm.at[1].at[sl], bbuf)
        pltpu.sync_copy(src_hbm.at[pl.ds(s * CPT, CPT)], sslab)
        pltpu.sync_copy(dst_hbm.at[pl.ds(s * CPT, CPT)], dslab)
        for r in range(RPT // RCH):
            pltpu.sync_copy(zbuf, acc.at[pl.ds(s * RPT + r * RCH, RCH)])
        plsc.subcore_barrier()

        bufs = [tna, tnb2]
        for k in range(1, KORD):
            alpha = 1.0 if k == 1 else 2.0
            beta = 0.0 if k == 1 else 1.0
            sub = bufs[(k - 1) % 2]

            def gat(ci, b):
                return pltpu.make_async_copy(
                    u_s.at[sslab.at[ci]], rows.at[b], gsems.at[b])

            def scat(ci, b):
                return pltpu.make_async_copy(
                    rows.at[b], acc.at[dslab.at[ci]], ssems.at[b])

            _ring_edge_loop(CPT, NL2, gat, scat)
            plsc.subcore_barrier()

            tcoef = th2_v[k, pl.ds(0, C)]
            for r in range(RPT // RCH):
                rsl = pl.ds(s * RPT + r * RCH, RCH)
                pltpu.sync_copy(acc.at[rsl], pbuf)
                pltpu.sync_copy(zbuf, acc.at[rsl])

                def rowop(i, carry):
                    t = r * RCH + i
                    p = pbuf[i, pl.ds(0, C)]
                    bb = bbuf[t, pl.ds(0, C)]
                    tn = (alpha * bb) * p
                    if beta != 0.0:
                        tn = tn - sub[t, pl.ds(0, C)]
                    sub[t, pl.ds(0, C)] = tn
                    unb[i, pl.ds(0, C)] = abuf[t, pl.ds(0, C)] * tn
                    ybuf[t, pl.ds(0, C)] = (ybuf[t, pl.ds(0, C)]
                                            + tcoef * tn)
                    return carry
                lax.fori_loop(0, RCH, rowop, 0)

                pltpu.sync_copy(unb, u_s.at[rsl])
            plsc.subcore_barrier()

        pltpu.sync_copy(ybuf, yout_hbm.at[pl.ds(s * RPT, RPT)])


_PBLK = 640


def _prescale_body(deg_ref, x_ref, u0_ref, ab16_ref):
    dg = deg_ref[...]
    a16 = lax.rsqrt(jnp.maximum(dg[0], 1.0))
    b16 = -lax.rsqrt(jnp.maximum(dg[1], 1.0))
    ab16_ref[0] = a16
    ab16_ref[1] = b16
    a1 = a16[:, :1]
    x = x_ref[...]
    u0_ref[0] = x[:, :W1SC] * a1
    u0_ref[1] = x[:, W1SC:] * a1


def _prescale(deg16, x_pad):
    return pl.pallas_call(
        _prescale_body,
        grid=(N_PAD // _PBLK,),
        in_specs=[
            pl.BlockSpec((2, _PBLK, 16), lambda i: (0, i, 0)),
            pl.BlockSpec((_PBLK, D), lambda i: (i, 0)),
        ],
        out_specs=[
            pl.BlockSpec((2, _PBLK, W1SC), lambda i: (0, i, 0)),
            pl.BlockSpec((2, _PBLK, 16), lambda i: (0, i, 0)),
        ],
        out_shape=[
            jax.ShapeDtypeStruct((NC, N_PAD, W1SC), _f32),
            jax.ShapeDtypeStruct((2, N_PAD, 16), _f32),
        ],
    )(deg16, x_pad)


@functools.lru_cache(None)
def _combine(alpha, beta):
    def body(p_ref, tp_ref, ab_ref, tn_ref, un_ref):
        p = p_ref[...]
        a = ab_ref[0][:, :1][None]
        b = ab_ref[1][:, :1][None]
        tn = (alpha * b) * p
        if beta != 0.0:
            tn = tn - beta * tp_ref[...]
        tn_ref[...] = tn
        un_ref[...] = a * tn

    def call(p, t_prev, ab16):
        return pl.pallas_call(
            body,
            grid=(N_PAD // _PBLK,),
            in_specs=[
                pl.BlockSpec((2, _PBLK, W1SC), lambda i: (0, i, 0)),
                pl.BlockSpec((2, _PBLK, W1SC), lambda i: (0, i, 0)),
                pl.BlockSpec((2, _PBLK, 16), lambda i: (0, i, 0)),
            ],
            out_specs=[
                pl.BlockSpec((2, _PBLK, W1SC), lambda i: (0, i, 0)),
                pl.BlockSpec((2, _PBLK, W1SC), lambda i: (0, i, 0)),
            ],
            out_shape=[
                jax.ShapeDtypeStruct((NC, N_PAD, W1SC), _f32),
                jax.ShapeDtypeStruct((NC, N_PAD, W1SC), _f32),
            ],
        )(p, t_prev, ab16)
    return call


_MBLK = 1024


def _mm1_body(t_ref, w_ref, thr_ref, y_ref):
    w = w_ref[...]
    acc = jnp.zeros((_MBLK, HEADS * HID), _f32)
    for k in range(KORD):
        wk = w * thr_ref[k, :][None, :]
        acc = acc + jnp.dot(t_ref[k], wk, preferred_element_type=_f32)
    y_ref[...] = acc


def _mm1(tcat, w1r, thr):
    return pl.pallas_call(
        _mm1_body,
        grid=(N_PAD // _MBLK,),
        in_specs=[
            pl.BlockSpec((KORD, _MBLK, D), lambda i: (0, i, 0)),
            pl.BlockSpec((D, HEADS * HID), lambda i: (0, 0)),
            pl.BlockSpec((KORD, HEADS * HID), lambda i: (0, 0)),
        ],
        out_specs=pl.BlockSpec((_MBLK, HEADS * HID), lambda i: (i, 0)),
        out_shape=jax.ShapeDtypeStruct((N_PAD, HEADS * HID), _f32),
    )(tcat, w1r, thr)


_M2BLK = 2048


def _mm2_body(y_ref, w2_ref, th2_ref, ab16_ref, g_ref, u0_ref, y0_ref,
              th2x_ref):
    g = jnp.dot(y_ref[...], w2_ref[...], preferred_element_type=_f32)
    g_ref[...] = g
    u0_ref[...] = ab16_ref[0] * g
    y0_ref[...] = g * th2_ref[0, 0:1][None, :]
    th2x_ref[...] = jnp.broadcast_to(
        th2_ref[...].reshape(KORD, 1), (KORD, C))


def _mm2(y, w2v, th2, ab16):
    return pl.pallas_call(
        _mm2_body,
        grid=(N_PAD // _M2BLK,),
        in_specs=[
            pl.BlockSpec((_M2BLK, HEADS * HID), lambda i: (i, 0)),
            pl.BlockSpec((HEADS * HID, C), lambda i: (0, 0)),
            pl.BlockSpec((1, KORD), lambda i: (0, 0)),
            pl.BlockSpec((2, _M2BLK, 16), lambda i: (0, i, 0)),
        ],
        out_specs=[
            pl.BlockSpec((_M2BLK, C), lambda i: (i, 0)),
            pl.BlockSpec((_M2BLK, C), lambda i: (i, 0)),
            pl.BlockSpec((_M2BLK, C), lambda i: (i, 0)),
            pl.BlockSpec((KORD, C), lambda i: (0, 0)),
        ],
        out_shape=[
            jax.ShapeDtypeStruct((N_PAD, C), _f32),
            jax.ShapeDtypeStruct((N_PAD, C), _f32),
            jax.ShapeDtypeStruct((N_PAD, C), _f32),
            jax.ShapeDtypeStruct((KORD, C), _f32),
        ],
    )(y, w2v, th2, ab16)


def _act_body(y_ref, o_ref):
    v = y_ref[...]
    e = jnp.where(v > 0, v, jnp.exp(jnp.minimum(v, 0.0)) - 1.0)
    m = jnp.max(e, axis=1, keepdims=True)
    z = e - m
    lse = jnp.log(jnp.sum(jnp.exp(z), axis=1, keepdims=True))
    o_ref[...] = z - lse


def _act(y):
    return pl.pallas_call(
        _act_body,
        grid=(N_PAD // _M2BLK,),
        in_specs=[pl.BlockSpec((_M2BLK, C), lambda i: (i, 0))],
        out_specs=pl.BlockSpec((_M2BLK, C), lambda i: (i, 0)),
        out_shape=jax.ShapeDtypeStruct((N_PAD, C), _f32),
    )(y)


def kernel(x, edge_index, theta1, W1, theta2, W2):
    src = edge_index[0]
    dst = edge_index[1]
    pad_i = jnp.full((E_PAD - E,), PAD_ROW, jnp.int32)
    src_p = jnp.concatenate([src, pad_i]).reshape(NCHUNK, B)
    dst_p = jnp.concatenate([dst, pad_i]).reshape(NCHUNK, B)
    x_pad = jnp.pad(x, ((0, N_PAD - N), (0, 0)))

    ones16 = jnp.ones((B, 16), _f32)
    z16r = jnp.zeros((RPT, 16), _f32)
    z64 = jnp.zeros((RPT, W1SC), _f32)

    deg16 = _deg_kernel(src_p, dst_p, ones16, z16r)
    u0, ab16 = _prescale(deg16, x_pad)

    x_split = x_pad.reshape(N_PAD, NC, W1SC).transpose(1, 0, 2)
    ts = [x_split]
    u = u0
    for k in range(1, KORD):
        al, be = (1.0, 0.0) if k == 1 else (2.0, 1.0)
        t_pp = ts[-2] if k >= 2 else ts[-1]
        p = _l1_gather_scatter(u, src_p, dst_p, z64)
        t_new, u_next = _combine(al, be)(p, t_pp, ab16)
        ts.append(t_new)
        u = u_next

    tcat = jnp.stack(
        [x_pad] + [jnp.concatenate([t[0], t[1]], axis=1) for t in ts[1:]])
    w1r = jnp.transpose(W1, (1, 0, 2)).reshape(D, HEADS * HID)
    thr = jnp.repeat(theta1.T, HID, axis=1)
    y1 = _mm1(tcat, w1r, thr)

    g, u0p, y0, th2x = _mm2(y1, W2[0], theta2, ab16)
    yacc = _l2_fused(g, u0p, y0, th2x, ab16, src_p, dst_p, z16r)

    out = _act(yacc)
    return out[:N]


# speedup vs baseline: 18.2798x; 1.6786x over previous
import functools

import jax
import jax.numpy as jnp
from jax import lax
from jax.experimental import pallas as pl
from jax.experimental.pallas import tpu as pltpu
from jax.experimental.pallas import tpu_sc as plsc

N = 10000
E = 320000
D = 128
HID = 64
HEADS = 4
KORD = 8
C = 16

NT = 16
NC = 2
NW = NT * NC
N_PAD = 10240
RPT = N_PAD // NT
PAD_ROW = N
B = 128
E_PAD = 327680
NCHUNK = E_PAD // B
EPW = E_PAD // NW
CPW = EPW // B
EPT = E_PAD // NT
CPT = EPT // B
RCH = 128
NB1 = 3
NB2 = 4
NL2 = 2


def _ring_edge_loop(nchunks, nb, gat, scat):
    def step(ci, carry):
        b = lax.rem(ci, nb)

        @pl.when(ci >= nb)
        def _():
            scat(ci - nb, b).wait()
        gat(ci, b).start()

        @pl.when(ci >= 1)
        def _():
            b1 = lax.rem(ci - 1, nb)
            gat(ci - 1, b1).wait()
            scat(ci - 1, b1).start(add=True)
        return carry
    lax.fori_loop(0, nchunks, step, 0)
    last = nchunks - 1
    gat(last, last % nb).wait()
    scat(last, last % nb).start(add=True)
    for j in range(max(last - nb + 1, 0), last + 1):
        scat(j, j % nb).wait()


_f32 = jnp.float32


def _mesh():
    return plsc.VectorSubcoreMesh(core_axis_name="c", subcore_axis_name="s")


_SC_PARAMS = pltpu.CompilerParams(use_tc_tiling_on_sc=False)


@functools.partial(
    pl.kernel,
    out_type=jax.ShapeDtypeStruct((2, N_PAD, 16), _f32),
    mesh=_mesh(),
    compiler_params=_SC_PARAMS,
    scratch_types=[
        pltpu.VMEM((CPT, B), jnp.int32),
        pltpu.VMEM((B, 16), _f32),
        pltpu.VMEM((RPT, 16), _f32),
        pltpu.VMEM_SHARED((N_PAD, 16), _f32),
        pltpu.SemaphoreType.DMA((NB2,)),
    ],
)
def _deg_kernel(src_hbm, dst_hbm, ones_hbm, z16_hbm, out_hbm, slab, ones_v,
                bounce, acc, sems):
    c = lax.axis_index("c")
    s = lax.axis_index("s")
    pltpu.sync_copy(z16_hbm, bounce)
    pltpu.sync_copy(bounce, acc.at[pl.ds(s * RPT, RPT)])
    pltpu.sync_copy(ones_hbm, ones_v)

    @pl.when(c == 0)
    def _():
        pltpu.sync_copy(src_hbm.at[pl.ds(s * CPT, CPT)], slab)

    @pl.when(c == 1)
    def _():
        pltpu.sync_copy(dst_hbm.at[pl.ds(s * CPT, CPT)], slab)

    plsc.subcore_barrier()

    def scat(ci, b):
        return pltpu.make_async_copy(ones_v, acc.at[slab.at[ci]], sems.at[b])

    def step(ci, carry):
        b = lax.rem(ci, NB2)

        @pl.when(ci >= NB2)
        def _():
            scat(ci - NB2, b).wait()
        scat(ci, b).start(add=True)
        return carry
    lax.fori_loop(0, CPT, step, 0)
    for t in range(NB2):
        scat(CPT - NB2 + t, (CPT - NB2 + t) % NB2).wait()

    plsc.subcore_barrier()
    pltpu.sync_copy(acc.at[pl.ds(s * RPT, RPT)], bounce)
    pltpu.sync_copy(bounce, out_hbm.at[c].at[pl.ds(s * RPT, RPT)])


W1SC = 64

@functools.partial(
    pl.kernel,
    out_type=jax.ShapeDtypeStruct((NC, N_PAD, W1SC), _f32),
    mesh=_mesh(),
    compiler_params=_SC_PARAMS,
    scratch_types=[
        pltpu.VMEM((CPT // 2, B), jnp.int32),
        pltpu.VMEM((CPT // 2, B), jnp.int32),
        pltpu.VMEM((NB1, B, W1SC), _f32),
        pltpu.SemaphoreType.DMA((NB1,)),
        pltpu.SemaphoreType.DMA((NB1,)),
        pltpu.VMEM_SHARED((N_PAD, W1SC), _f32),
        pltpu.VMEM_SHARED((N_PAD, W1SC), _f32),
    ],
)
def _l1_gather_scatter(u_hbm, src_hbm, dst_hbm, z64_hbm, out_hbm,
                       sslab, dslab, rows, gsems, ssems, u_sp, acc):
    c = lax.axis_index("c")
    s = lax.axis_index("s")
    pltpu.sync_copy(z64_hbm.at[pl.ds(0, RCH)], rows.at[0])
    for r in range(RPT // RCH):
        rsl = pl.ds(s * RPT + r * RCH, RCH)
        pltpu.sync_copy(rows.at[0], acc.at[rsl])
        pltpu.sync_copy(u_hbm.at[c].at[rsl], rows.at[1])
        pltpu.sync_copy(rows.at[1], u_sp.at[rsl])
    plsc.subcore_barrier()

    def gat(ci, b):
        return pltpu.make_async_copy(
            u_sp.at[sslab.at[ci]], rows.at[b], gsems.at[b])

    def scat(ci, b):
        return pltpu.make_async_copy(
            rows.at[b], acc.at[dslab.at[ci]], ssems.at[b])

    for ph in range(2):
        hb = s * CPT + ph * (CPT // 2)
        pltpu.sync_copy(src_hbm.at[pl.ds(hb, CPT // 2)], sslab)
        pltpu.sync_copy(dst_hbm.at[pl.ds(hb, CPT // 2)], dslab)
        _ring_edge_loop(CPT // 2, NB1, gat, scat)
    plsc.subcore_barrier()

    for r in range(RPT // RCH):
        rsl = pl.ds(s * RPT + r * RCH, RCH)
        pltpu.sync_copy(acc.at[rsl], rows.at[0])
        pltpu.sync_copy(rows.at[0], out_hbm.at[c].at[rsl])


@functools.partial(
    pl.kernel,
    out_type=jax.ShapeDtypeStruct((N_PAD, C), _f32),
    mesh=_mesh(),
    compiler_params=_SC_PARAMS,
    scratch_types=[
        pltpu.VMEM((CPT, B), jnp.int32),
        pltpu.VMEM((CPT, B), jnp.int32),
        pltpu.VMEM((8, C), _f32),
        pltpu.VMEM((RCH, C), _f32),
        pltpu.VMEM((RCH, C), _f32),
        pltpu.VMEM((RCH, C), _f32),
        pltpu.VMEM((RPT, C), _f32),
        pltpu.VMEM((RPT, C), _f32),
        pltpu.VMEM((RPT, C), _f32),
        pltpu.VMEM((RPT, C), _f32),
        pltpu.VMEM((RPT, C), _f32),
        pltpu.VMEM((NL2, B, C), _f32),
        pltpu.SemaphoreType.DMA((NL2,)),
        pltpu.SemaphoreType.DMA((NL2,)),
        pltpu.VMEM_SHARED((N_PAD, C), _f32),
        pltpu.VMEM_SHARED((N_PAD, C), _f32),
    ],
)
def _l2_fused(g_hbm, u0_hbm, y0_hbm, th2_hbm, ab16_hbm, src_hbm, dst_hbm,
              z16_hbm, yout_hbm,
              sslab, dslab, th2_v, pbuf, zbuf, unb, tna, tnb2,
              abuf, bbuf, ybuf, rows, gsems, ssems, u_s, acc):
    c = lax.axis_index("c")
    s = lax.axis_index("s")

    @pl.when(c == 0)
    def _():
        sl = pl.ds(s * RPT, RPT)
        pltpu.sync_copy(th2_hbm, th2_v)
        pltpu.sync_copy(z16_hbm.at[pl.ds(0, RCH)], zbuf)
        pltpu.sync_copy(g_hbm.at[sl], tna)
        pltpu.sync_copy(g_hbm.at[sl], tnb2)
        pltpu.sync_copy(u0_hbm.at[sl], ybuf)
        pltpu.sync_copy(ybuf, u_s.at[sl])
        pltpu.sync_copy(y0_hbm.at[sl], ybuf)
        pltpu.sync_copy(ab16_hbm.at[0].at[sl], abuf)
        pltpu.sync_copy(ab16_hbm.at[1].at[sl], bbuf)
        pltpu.sync_copy(src_hbm.at[pl.ds(s * CPT, CPT)], sslab)
        pltpu.sync_copy(dst_hbm.at[pl.ds(s * CPT, CPT)], dslab)
        for r in range(RPT // RCH):
            pltpu.sync_copy(zbuf, acc.at[pl.ds(s * RPT + r * RCH, RCH)])
        plsc.subcore_barrier()

        bufs = [tna, tnb2]
        for k in range(1, KORD):
            alpha = 1.0 if k == 1 else 2.0
            beta = 0.0 if k == 1 else 1.0
            sub = bufs[(k - 1) % 2]

            def gat(ci, b):
                return pltpu.make_async_copy(
                    u_s.at[sslab.at[ci]], rows.at[b], gsems.at[b])

            def scat(ci, b):
                return pltpu.make_async_copy(
                    rows.at[b], acc.at[dslab.at[ci]], ssems.at[b])

            _ring_edge_loop(CPT, NL2, gat, scat)
            plsc.subcore_barrier()

            tcoef = th2_v[k, pl.ds(0, C)]
            for r in range(RPT // RCH):
                rsl = pl.ds(s * RPT + r * RCH, RCH)
                pltpu.sync_copy(acc.at[rsl], pbuf)
                pltpu.sync_copy(zbuf, acc.at[rsl])

                def rowop(i, carry):
                    t = r * RCH + i
                    p = pbuf[i, pl.ds(0, C)]
                    bb = bbuf[t, pl.ds(0, C)]
                    tn = (alpha * bb) * p
                    if beta != 0.0:
                        tn = tn - sub[t, pl.ds(0, C)]
                    sub[t, pl.ds(0, C)] = tn
                    unb[i, pl.ds(0, C)] = abuf[t, pl.ds(0, C)] * tn
                    ybuf[t, pl.ds(0, C)] = (ybuf[t, pl.ds(0, C)]
                                            + tcoef * tn)
                    return carry
                lax.fori_loop(0, RCH, rowop, 0)

                pltpu.sync_copy(unb, u_s.at[rsl])
            plsc.subcore_barrier()

        pltpu.sync_copy(ybuf, yout_hbm.at[pl.ds(s * RPT, RPT)])


_PBLK = 640


def _prescale_body(deg_ref, x_ref, u0_ref, ab16_ref):
    dg = deg_ref[...]
    a16 = lax.rsqrt(jnp.maximum(dg[0], 1.0))
    b16 = -lax.rsqrt(jnp.maximum(dg[1], 1.0))
    ab16_ref[0] = a16
    ab16_ref[1] = b16
    a1 = a16[:, :1]
    x = x_ref[...]
    u0_ref[0] = x[:, :W1SC] * a1
    u0_ref[1] = x[:, W1SC:] * a1


def _prescale(deg16, x_pad):
    return pl.pallas_call(
        _prescale_body,
        grid=(N_PAD // _PBLK,),
        in_specs=[
            pl.BlockSpec((2, _PBLK, 16), lambda i: (0, i, 0)),
            pl.BlockSpec((_PBLK, D), lambda i: (i, 0)),
        ],
        out_specs=[
            pl.BlockSpec((2, _PBLK, W1SC), lambda i: (0, i, 0)),
            pl.BlockSpec((2, _PBLK, 16), lambda i: (0, i, 0)),
        ],
        out_shape=[
            jax.ShapeDtypeStruct((NC, N_PAD, W1SC), _f32),
            jax.ShapeDtypeStruct((2, N_PAD, 16), _f32),
        ],
    )(deg16, x_pad)


@functools.lru_cache(None)
def _combine(alpha, beta):
    def body(p_ref, tp_ref, ab_ref, tn_ref, un_ref):
        p = p_ref[...]
        a = ab_ref[0][:, :1][None]
        b = ab_ref[1][:, :1][None]
        tn = (alpha * b) * p
        if beta != 0.0:
            tn = tn - beta * tp_ref[...]
        tn_ref[...] = tn
        un_ref[...] = a * tn

    def call(p, t_prev, ab16):
        return pl.pallas_call(
            body,
            grid=(N_PAD // _PBLK,),
            in_specs=[
                pl.BlockSpec((2, _PBLK, W1SC), lambda i: (0, i, 0)),
                pl.BlockSpec((2, _PBLK, W1SC), lambda i: (0, i, 0)),
                pl.BlockSpec((2, _PBLK, 16), lambda i: (0, i, 0)),
            ],
            out_specs=[
                pl.BlockSpec((2, _PBLK, W1SC), lambda i: (0, i, 0)),
                pl.BlockSpec((2, _PBLK, W1SC), lambda i: (0, i, 0)),
            ],
            out_shape=[
                jax.ShapeDtypeStruct((NC, N_PAD, W1SC), _f32),
                jax.ShapeDtypeStruct((NC, N_PAD, W1SC), _f32),
            ],
        )(p, t_prev, ab16)
    return call


_MBLK = 1024


def _mm1_body(t_ref, w_ref, thr_ref, y_ref):
    w = w_ref[...]
    acc = jnp.zeros((_MBLK, HEADS * HID), _f32)
    for k in range(KORD):
        wk = w * thr_ref[k, :][None, :]
        acc = acc + jnp.dot(t_ref[k], wk, preferred_element_type=_f32)
    y_ref[...] = acc


def _mm1(tcat, w1r, thr):
    return pl.pallas_call(
        _mm1_body,
        grid=(N_PAD // _MBLK,),
        in_specs=[
            pl.BlockSpec((KORD, _MBLK, D), lambda i: (0, i, 0)),
            pl.BlockSpec((D, HEADS * HID), lambda i: (0, 0)),
            pl.BlockSpec((KORD, HEADS * HID), lambda i: (0, 0)),
        ],
        out_specs=pl.BlockSpec((_MBLK, HEADS * HID), lambda i: (i, 0)),
        out_shape=jax.ShapeDtypeStruct((N_PAD, HEADS * HID), _f32),
    )(tcat, w1r, thr)


_M2BLK = 2048


def _mm2_body(y_ref, w2_ref, th2_ref, ab16_ref, g_ref, u0_ref, y0_ref,
              th2x_ref):
    g = jnp.dot(y_ref[...], w2_ref[...], preferred_element_type=_f32)
    g_ref[...] = g
    u0_ref[...] = ab16_ref[0] * g
    y0_ref[...] = g * th2_ref[0, 0:1][None, :]
    th2x_ref[...] = jnp.broadcast_to(
        th2_ref[...].reshape(KORD, 1), (KORD, C))


def _mm2(y, w2v, th2, ab16):
    return pl.pallas_call(
        _mm2_body,
        grid=(N_PAD // _M2BLK,),
        in_specs=[
            pl.BlockSpec((_M2BLK, HEADS * HID), lambda i: (i, 0)),
            pl.BlockSpec((HEADS * HID, C), lambda i: (0, 0)),
            pl.BlockSpec((1, KORD), lambda i: (0, 0)),
            pl.BlockSpec((2, _M2BLK, 16), lambda i: (0, i, 0)),
        ],
        out_specs=[
            pl.BlockSpec((_M2BLK, C), lambda i: (i, 0)),
            pl.BlockSpec((_M2BLK, C), lambda i: (i, 0)),
            pl.BlockSpec((_M2BLK, C), lambda i: (i, 0)),
            pl.BlockSpec((KORD, C), lambda i: (0, 0)),
        ],
        out_shape=[
            jax.ShapeDtypeStruct((N_PAD, C), _f32),
            jax.ShapeDtypeStruct((N_PAD, C), _f32),
            jax.ShapeDtypeStruct((N_PAD, C), _f32),
            jax.ShapeDtypeStruct((KORD, C), _f32),
        ],
    )(y, w2v, th2, ab16)


def _act_body(y_ref, o_ref):
    v = y_ref[...]
    e = jnp.where(v > 0, v, jnp.exp(jnp.minimum(v, 0.0)) - 1.0)
    m = jnp.max(e, axis=1, keepdims=True)
    z = e - m
    lse = jnp.log(jnp.sum(jnp.exp(z), axis=1, keepdims=True))
    o_ref[...] = z - lse


def _act(y):
    return pl.pallas_call(
        _act_body,
        grid=(N_PAD // _M2BLK,),
        in_specs=[pl.BlockSpec((_M2BLK, C), lambda i: (i, 0))],
        out_specs=pl.BlockSpec((_M2BLK, C), lambda i: (i, 0)),
        out_shape=jax.ShapeDtypeStruct((N_PAD, C), _f32),
    )(y)


def kernel(x, edge_index, theta1, W1, theta2, W2):
    src = edge_index[0]
    dst = edge_index[1]
    pad_i = jnp.full((E_PAD - E,), PAD_ROW, jnp.int32)
    src_p = jnp.concatenate([src, pad_i]).reshape(NCHUNK, B)
    dst_p = jnp.concatenate([dst, pad_i]).reshape(NCHUNK, B)
    x_pad = jnp.pad(x, ((0, N_PAD - N), (0, 0)))

    ones16 = jnp.ones((B, 16), _f32)
    z16r = jnp.zeros((RPT, 16), _f32)
    z64 = jnp.zeros((RPT, W1SC), _f32)

    deg16 = _deg_kernel(src_p, dst_p, ones16, z16r)
    u0, ab16 = _prescale(deg16, x_pad)

    x_split = x_pad.reshape(N_PAD, NC, W1SC).transpose(1, 0, 2)
    ts = [x_split]
    u = u0
    for k in range(1, KORD):
        al, be = (1.0, 0.0) if k == 1 else (2.0, 1.0)
        t_pp = ts[-2] if k >= 2 else ts[-1]
        p = _l1_gather_scatter(u, src_p, dst_p, z64)
        t_new, u_next = _combine(al, be)(p, t_pp, ab16)
        ts.append(t_new)
        u = u_next

    tcat = jnp.stack(
        [x_pad] + [jnp.concatenate([t[0], t[1]], axis=1) for t in ts[1:]])
    w1r = jnp.transpose(W1, (1, 0, 2)).reshape(D, HEADS * HID)
    thr = jnp.repeat(theta1.T, HID, axis=1)
    y1 = _mm1(tcat, w1r, thr)

    g, u0p, y0, th2x = _mm2(y1, W2[0], theta2, ab16)
    yacc = _l2_fused(g, u0p, y0, th2x, ab16, src_p, dst_p, z16r)

    out = _act(yacc)
    return out[:N]
